# baseline (speedup 1.0000x reference)
import jax
import jax.numpy as jnp
from jax.experimental import pallas as pl


def _mlp_fused(x_ref, w1_ref, b1_ref, w2_ref, b2_ref, w3_ref, b3_ref, o_ref):
    dn = (((1,), (1,)), ((), ()))
    x = x_ref[...].astype(jnp.bfloat16)
    h = jax.lax.dot_general(x, w1_ref[...].astype(jnp.bfloat16), dn,
                            preferred_element_type=jnp.float32)
    h = jnp.maximum(h + b1_ref[...], 0.0).astype(jnp.bfloat16)
    h = jax.lax.dot_general(h, w2_ref[...].astype(jnp.bfloat16), dn,
                            preferred_element_type=jnp.float32)
    h = jnp.maximum(h + b2_ref[...], 0.0).astype(jnp.bfloat16)
    o = jax.lax.dot_general(h, w3_ref[...].astype(jnp.bfloat16), dn,
                            preferred_element_type=jnp.float32)
    o_ref[...] = o + b3_ref[...]


def kernel(x, W1, b1, W2, b2, W3, b3, t):
    del t
    B, D_OUT = x.shape[0], W3.shape[0]
    return pl.pallas_call(
        _mlp_fused,
        out_shape=jax.ShapeDtypeStruct((B, D_OUT), jnp.float32),
    )(x, W1, b1.reshape(1, -1), W2, b2.reshape(1, -1), W3, b3.reshape(1, -1))


# speedup vs baseline: 1.0166x; 1.0166x over previous
import jax
import jax.numpy as jnp
from jax.experimental import pallas as pl


def _mlp_fused(x_ref, w1_ref, b1_ref, w2_ref, b2_ref, w3_ref, b3_ref, o_ref):
    dn = (((1,), (1,)), ((), ()))
    h = jax.lax.dot_general(x_ref[...], w1_ref[...], dn,
                            preferred_element_type=jnp.float32)
    h = jnp.maximum(h + b1_ref[...], 0.0)
    h = jax.lax.dot_general(h, w2_ref[...], dn,
                            preferred_element_type=jnp.float32)
    h = jnp.maximum(h + b2_ref[...], 0.0)
    o = jax.lax.dot_general(h, w3_ref[...], dn,
                            preferred_element_type=jnp.float32)
    o_ref[...] = o + b3_ref[...]


def kernel(x, W1, b1, W2, b2, W3, b3, t):
    del t
    B, D_OUT = x.shape[0], W3.shape[0]
    return pl.pallas_call(
        _mlp_fused,
        out_shape=jax.ShapeDtypeStruct((B, D_OUT), jnp.float32),
    )(x, W1, b1.reshape(1, -1), W2, b2.reshape(1, -1), W3, b3.reshape(1, -1))
